# baseline (device time: 8206 ns/iter reference)
import jax
import jax.numpy as jnp
from jax import lax
from jax.experimental import pallas as pl
from jax.experimental.pallas import tpu as pltpu

N_CHUNKS = 2


def kernel(x):
    _, m, n2 = x.shape
    n = n2 // 2
    mc = m // N_CHUNKS

    def body(x_ref, out_ref, comm_ref, send_sems, recv_sems):
        my_x = lax.axis_index("x")
        my_y = lax.axis_index("y")
        peer_y = 1 - my_y

        barrier_sem = pltpu.get_barrier_semaphore()
        pl.semaphore_signal(
            barrier_sem,
            inc=1,
            device_id=(my_x, peer_y),
            device_id_type=pl.DeviceIdType.MESH,
        )

        def prestore(my_lo):
            out_ref[:, :] = x_ref[0, :, pl.ds(my_lo, n)]

        @pl.when(my_y == 0)
        def _():
            prestore(0)

        @pl.when(my_y == 1)
        def _():
            prestore(n)

        pl.semaphore_wait(barrier_sem, 1)

        def xchg(peer_lo):
            rdmas = []
            for c in range(N_CHUNKS):
                rdma = pltpu.make_async_remote_copy(
                    src_ref=x_ref.at[0, pl.ds(c * mc, mc), pl.ds(peer_lo, n)],
                    dst_ref=comm_ref.at[pl.ds(c * mc, mc)],
                    send_sem=send_sems.at[c],
                    recv_sem=recv_sems.at[c],
                    device_id=(my_x, peer_y),
                    device_id_type=pl.DeviceIdType.MESH,
                )
                rdma.start()
                rdmas.append(rdma)
            for c, rdma in enumerate(rdmas):
                rdma.wait()
                rows = pl.ds(c * mc, mc)
                out_ref[rows, :] = out_ref[rows, :] + comm_ref[rows, :]

        @pl.when(my_y == 0)
        def _():
            xchg(n)

        @pl.when(my_y == 1)
        def _():
            xchg(0)

    return pl.pallas_call(
        body,
        out_shape=jax.ShapeDtypeStruct((m, n), x.dtype),
        in_specs=[pl.BlockSpec(memory_space=pltpu.VMEM)],
        out_specs=pl.BlockSpec(memory_space=pltpu.VMEM),
        scratch_shapes=[
            pltpu.VMEM((m, n), x.dtype),
            pltpu.SemaphoreType.DMA((N_CHUNKS,)),
            pltpu.SemaphoreType.DMA((N_CHUNKS,)),
        ],
        compiler_params=pltpu.CompilerParams(collective_id=0),
    )(x)
